# Initial kernel scaffold; baseline (speedup 1.0000x reference)
import jax
import jax.numpy as jnp
from jax.experimental import pallas as pl


def kernel(xyz, params):
    raise NotImplementedError("write your pallas kernel here")


# speedup vs baseline: 3.0674x; 3.0674x over previous
import functools

import jax
import jax.numpy as jnp
from jax.experimental import pallas as pl
from jax.experimental.pallas import tpu as pltpu

_B, _N0 = 8, 2048

_SA_CFGS = [
    {"npoint": 512, "radii": [0.1, 0.2, 0.4], "nsamples": [32, 64, 128],
     "in_channel": 6, "mlps": [[32, 32, 64], [64, 64, 128], [64, 96, 128]]},
    {"npoint": 128, "radii": [0.4, 0.8], "nsamples": [64, 128],
     "in_channel": 320, "mlps": [[128, 128, 256], [128, 196, 256]]},
    {"npoint": 64, "radii": [0.2, 0.4], "nsamples": [16, 32],
     "in_channel": 512, "mlps": [[128, 128, 256], [128, 196, 256]]},
]

_GROUP_SBLK = {0: 64, 1: 64, 2: 64}
_MLP_SBLK = {0: 32, 1: 32, 2: 64}


def _fps_body(xyzT_ref, out_ref, *, npoint):
    x = xyzT_ref[...]
    b, _, n = x.shape
    iota_n = jax.lax.broadcasted_iota(jnp.int32, (b, 1, n), 2).astype(jnp.float32)
    iota_s = jax.lax.broadcasted_iota(jnp.int32, (1, 1, npoint), 2)

    def step(i, carry):
        dist, far, acc = carry
        oh = (iota_n == far).astype(jnp.float32)
        c = jnp.sum(x * oh, axis=2, keepdims=True)
        d = jnp.sum((x - c) ** 2, axis=1, keepdims=True)
        dist = jnp.minimum(dist, d)
        m = jnp.max(dist, axis=2, keepdims=True)
        far2 = jnp.min(jnp.where(dist == m, iota_n, jnp.float32(n)),
                       axis=2, keepdims=True)
        ohs = (iota_s == i).astype(jnp.float32)
        acc = acc + c * ohs
        return dist, far2, acc

    init = (jnp.full((b, 1, n), 1e10, jnp.float32),
            jnp.zeros((b, 1, 1), jnp.float32),
            jnp.zeros((b, 3, npoint), jnp.float32))
    _, _, acc = jax.lax.fori_loop(0, npoint, step, init)
    out_ref[...] = acc


def _fps(xyzT, npoint):
    b, _, n = xyzT.shape
    return pl.pallas_call(
        functools.partial(_fps_body, npoint=npoint),
        out_shape=jax.ShapeDtypeStruct((b, 3, npoint), jnp.float32),
    )(xyzT)


def _group_body(data_ref, xyzT_ref, new3_ref, lt_ref, w1_ref, w1x_ref,
                b1_ref, y_ref, st_ref, rm1_ref, cnt_ref, *, r2, k, s_blk):
    b = pl.program_id(0)
    sb = pl.program_id(1)
    data = data_ref[0]
    xyzT = xyzT_ref[0]
    new3 = new3_ref[0]
    n = data.shape[0]
    c1 = w1_ref.shape[1]

    d2 = (-2.0 * jnp.dot(new3, xyzT, preferred_element_type=jnp.float32)
          + jnp.sum(new3 * new3, axis=1, keepdims=True)
          + jnp.sum(xyzT * xyzT, axis=0, keepdims=True))
    mask = (d2 <= r2).astype(jnp.float32)
    rank = jnp.dot(mask, lt_ref[...], preferred_element_type=jnp.float32, precision=jax.lax.Precision.HIGHEST)
    rm1_ref[...] = jnp.where(mask > 0.0, rank - 1.0, -1.0)
    cnt_ref[...] = jnp.max(rank, axis=1, keepdims=True)

    iota_k = jax.lax.broadcasted_iota(jnp.int32, (k, 1), 0).astype(jnp.float32)
    w1 = w1_ref[...]
    w1x = w1x_ref[...]
    b1 = b1_ref[...]

    def body(s, carry):
        ssum, ssq = carry
        row = rm1_ref[pl.ds(s, 1), :]
        mt = (jnp.abs(row - iota_k) < 0.5).astype(jnp.float32)
        g = jnp.dot(mt, data, preferred_element_type=jnp.float32, precision=jax.lax.Precision.HIGHEST)
        cnt = cnt_ref[pl.ds(s, 1), :]
        row0 = jnp.where(cnt > 0.5, g[0:1, :], data[n - 1:n, :])
        g = jnp.where(iota_k + 0.5 < cnt, g, row0)
        crow = new3_ref[0, pl.ds(s, 1), :]
        cpad = jnp.concatenate(
            [jnp.zeros((1, g.shape[1] - 3), jnp.float32), crow], axis=1)
        y = jnp.dot(g - cpad, w1, preferred_element_type=jnp.float32) + b1
        y_ref[0, pl.ds(s, 1)] = y[None]
        ssum = ssum + jnp.sum(y, axis=0, keepdims=True)
        ssq = ssq + jnp.sum(y * y, axis=0, keepdims=True)
        return ssum, ssq

    z = jnp.zeros((1, c1), jnp.float32)
    ssum, ssq = jax.lax.fori_loop(0, s_blk, body, (z, z))
    part = jnp.concatenate([ssum, ssq, jnp.zeros((6, c1), jnp.float32)], 0)
    first = jnp.logical_and(b == 0, sb == 0)

    @pl.when(first)
    def _():
        st_ref[...] = part

    @pl.when(jnp.logical_not(first))
    def _():
        st_ref[...] = st_ref[...] + part


def _group_mm1(data, xyzT, new3, lt, w1, w1x, b1, r2, k, s_blk):
    b, n, cin3 = data.shape
    s = new3.shape[1]
    c1 = w1.shape[1]
    return pl.pallas_call(
        functools.partial(_group_body, r2=r2, k=k, s_blk=s_blk),
        grid=(b, s // s_blk),
        in_specs=[
            pl.BlockSpec((1, n, cin3), lambda i, j: (i, 0, 0)),
            pl.BlockSpec((1, 3, n), lambda i, j: (i, 0, 0)),
            pl.BlockSpec((1, s_blk, 3), lambda i, j: (i, j, 0)),
            pl.BlockSpec((n, n), lambda i, j: (0, 0)),
            pl.BlockSpec((cin3, c1), lambda i, j: (0, 0)),
            pl.BlockSpec((3, c1), lambda i, j: (0, 0)),
            pl.BlockSpec((1, c1), lambda i, j: (0, 0)),
        ],
        out_specs=[
            pl.BlockSpec((1, s_blk, k, c1), lambda i, j: (i, j, 0, 0)),
            pl.BlockSpec((8, c1), lambda i, j: (0, 0)),
        ],
        out_shape=[
            jax.ShapeDtypeStruct((b, s, k, c1), jnp.float32),
            jax.ShapeDtypeStruct((8, c1), jnp.float32),
        ],
        scratch_shapes=[
            pltpu.VMEM((s_blk, n), jnp.float32),
            pltpu.VMEM((s_blk, 1), jnp.float32),
        ],
    )(data, xyzT, new3, lt, w1, w1x, b1)


def _mlp_body(y_ref, sc_ref, sh_ref, w_ref, b_ref, o_ref, st_ref):
    b = pl.program_id(0)
    sb = pl.program_id(1)
    x = y_ref[0]
    s_blk, k, cp = x.shape
    cn = w_ref.shape[1]
    x = jnp.maximum(x * sc_ref[...] + sh_ref[...], 0.0)
    xf = x.reshape(s_blk * k, cp)
    yo = jnp.dot(xf, w_ref[...], preferred_element_type=jnp.float32) + b_ref[...]
    o_ref[0] = yo.reshape(s_blk, k, cn)
    ssum = jnp.sum(yo, axis=0, keepdims=True)
    ssq = jnp.sum(yo * yo, axis=0, keepdims=True)
    part = jnp.concatenate([ssum, ssq, jnp.zeros((6, cn), jnp.float32)], 0)
    first = jnp.logical_and(b == 0, sb == 0)

    @pl.when(first)
    def _():
        st_ref[...] = part

    @pl.when(jnp.logical_not(first))
    def _():
        st_ref[...] = st_ref[...] + part


def _mlp(y, sc, sh, w, bb, s_blk):
    b, s, k, cp = y.shape
    cn = w.shape[1]
    return pl.pallas_call(
        _mlp_body,
        grid=(b, s // s_blk),
        in_specs=[
            pl.BlockSpec((1, s_blk, k, cp), lambda i, j: (i, j, 0, 0)),
            pl.BlockSpec((1, cp), lambda i, j: (0, 0)),
            pl.BlockSpec((1, cp), lambda i, j: (0, 0)),
            pl.BlockSpec((cp, cn), lambda i, j: (0, 0)),
            pl.BlockSpec((1, cn), lambda i, j: (0, 0)),
        ],
        out_specs=[
            pl.BlockSpec((1, s_blk, k, cn), lambda i, j: (i, j, 0, 0)),
            pl.BlockSpec((8, cn), lambda i, j: (0, 0)),
        ],
        out_shape=[
            jax.ShapeDtypeStruct((b, s, k, cn), jnp.float32),
            jax.ShapeDtypeStruct((8, cn), jnp.float32),
        ],
    )(y, sc, sh, w, bb)


def _pool_body(y_ref, sc_ref, sh_ref, o_ref):
    x = y_ref[0]
    x = jnp.maximum(x * sc_ref[...] + sh_ref[...], 0.0)
    o_ref[0] = jnp.max(x, axis=1)


def _pool(y, sc, sh, s_blk):
    b, s, k, c = y.shape
    return pl.pallas_call(
        _pool_body,
        grid=(b, s // s_blk),
        in_specs=[
            pl.BlockSpec((1, s_blk, k, c), lambda i, j: (i, j, 0, 0)),
            pl.BlockSpec((1, c), lambda i, j: (0, 0)),
            pl.BlockSpec((1, c), lambda i, j: (0, 0)),
        ],
        out_specs=pl.BlockSpec((1, s_blk, c), lambda i, j: (i, j, 0)),
        out_shape=jax.ShapeDtypeStruct((b, s, c), jnp.float32),
    )(y, sc, sh)


def _bn_coeffs(st, cnt, gamma, beta):
    ssum, ssq = st[0], st[1]
    mean = ssum / cnt
    var = ssq / cnt - mean * mean
    inv = jax.lax.rsqrt(var + 1e-5)
    scale = gamma * inv
    shift = beta - mean * scale
    return scale[None, :], shift[None, :]


def _sa_layer(points_nc, xyzT, cfg, layer_params, li):
    b, n, _ = points_nc.shape
    s = cfg["npoint"]
    newT = _fps(xyzT, s)
    new3 = jnp.transpose(newT, (0, 2, 1))
    data = jnp.concatenate([points_nc, jnp.transpose(xyzT, (0, 2, 1))], -1)
    ii = jax.lax.broadcasted_iota(jnp.float32, (n, n), 0)
    jj = jax.lax.broadcasted_iota(jnp.float32, (n, n), 1)
    lt = (ii <= jj).astype(jnp.float32)
    gs = _GROUP_SBLK[li]
    ms = _MLP_SBLK[li]
    pooled = []
    for radius, k, branch in zip(cfg["radii"], cfg["nsamples"], layer_params):
        w1 = jnp.transpose(branch[0]["W"])
        y, st = _group_mm1(data, xyzT, new3, lt, w1, w1[-3:],
                           branch[0]["b"][None], radius * radius, k, gs)
        cnt = float(b * s * k)
        for p_prev, p_next in zip(branch[:-1], branch[1:]):
            sc, sh = _bn_coeffs(st, cnt, p_prev["gamma"], p_prev["beta"])
            y, st = _mlp(y, sc, sh, jnp.transpose(p_next["W"]),
                         p_next["b"][None], ms)
        sc, sh = _bn_coeffs(st, cnt, branch[-1]["gamma"], branch[-1]["beta"])
        pooled.append(_pool(y, sc, sh, ms))
    return newT, jnp.concatenate(pooled, -1)


def kernel(xyz, params):
    l0_xyz = xyz[:, :3, :]
    points0 = jnp.transpose(xyz, (0, 2, 1))
    l1T, p1 = _sa_layer(points0, l0_xyz, _SA_CFGS[0], params[0], 0)
    l2T, p2 = _sa_layer(p1, l1T, _SA_CFGS[1], params[1], 1)
    l3T, p3 = _sa_layer(p2, l2T, _SA_CFGS[2], params[2], 2)
    return (l0_xyz, xyz,
            l1T, jnp.transpose(p1, (0, 2, 1)),
            l2T, jnp.transpose(p2, (0, 2, 1)),
            l3T, jnp.transpose(p3, (0, 2, 1)))
